# baseline (speedup 1.0000x reference)
import jax
import jax.numpy as jnp
from jax import lax
from jax.experimental import pallas as pl
from jax.experimental.pallas import tpu as pltpu
from jax.experimental.pallas import tpu_sc as plsc

NUM_NODES = 10000
NUM_FACES = 320000
NUM_CH = 128

_NC = 2
_NS = 16
_NW = _NC * _NS
_CHUNK = 80
_NCHUNKS = NUM_FACES // _CHUNK
_NBUF = 3
_MAX_ITERS = _NCHUNKS // _NW
_ROWS_PER_TILE = 624
_TAIL_ROWS = NUM_NODES - _NS * _ROWS_PER_TILE


def _sc_body(phi_hbm, fn_hbm, sums_hbm, cnts_hbm,
             fb0, fb1, fb2, ib0, ib1, ib2, cntloc, acc,
             lsem0, lsem1, lsem2, ssem0, ssem1, ssem2):
  c = lax.axis_index("core")
  s = lax.axis_index("subcore")
  wid = s * _NC + c

  fbufs = (fb0, fb1, fb2)
  ibufs = (ib0, ib1, ib2)
  lsems = (lsem0, lsem1, lsem2)
  ssems = (ssem0, ssem1, ssem2)

  z16 = jnp.zeros((16,), jnp.float32)
  ones16 = jnp.ones((16,), jnp.float32)

  def start_load(chunk, q):
    base = chunk * _CHUNK
    pltpu.async_copy(phi_hbm.at[pl.ds(base, _CHUNK)], fbufs[q], lsems[q])
    pltpu.async_copy(fn_hbm.at[:, pl.ds(base, _CHUNK)], ibufs[q], lsems[q])

  def wait_load(chunk, p):
    base = chunk * _CHUNK
    pltpu.make_async_copy(phi_hbm.at[pl.ds(base, _CHUNK)], fbufs[p],
                          lsems[p]).wait()
    pltpu.make_async_copy(fn_hbm.at[:, pl.ds(base, _CHUNK)], ibufs[p],
                          lsems[p]).wait()

  def wait_scatter(q):
    pltpu.make_async_copy(fbufs[q], acc.at[ibufs[q].at[0]], ssems[q]).wait()
    pltpu.make_async_copy(fbufs[q], acc.at[ibufs[q].at[1]], ssems[q]).wait()

  start_load(_NW + wid, 1)

  @pl.loop(0, _CHUNK)
  def _(i):
    @pl.loop(0, NUM_CH // 16)
    def _(j):
      fb0[i, pl.ds(j * 16, 16)] = z16

  @pl.loop(0, NUM_NODES // 16)
  def _(i):
    cntloc[pl.ds(i * 16, 16)] = z16

  row0 = s * _ROWS_PER_TILE
  tail0 = _NS * _ROWS_PER_TILE
  @pl.loop(0, _ROWS_PER_TILE // _CHUNK)
  def _(k):
    pltpu.sync_copy(fb0, acc.at[pl.ds(row0 + k * _CHUNK, _CHUNK)])
  rem = _ROWS_PER_TILE % _CHUNK
  if rem:
    base = row0 + (_ROWS_PER_TILE // _CHUNK) * _CHUNK
    pltpu.sync_copy(fb0.at[pl.ds(0, rem)], acc.at[pl.ds(base, rem)])

  @pl.when(s == _NS - 1)
  def _():
    pltpu.sync_copy(fb0.at[pl.ds(0, _TAIL_ROWS)],
                    acc.at[pl.ds(tail0, _TAIL_ROWS)])

  start_load(wid, 0)

  plsc.subcore_barrier()

  def _it(i, p, q, first=False):
    chunk = i * _NW + wid

    @pl.when(chunk < _NCHUNKS)
    def _():
      fb = fbufs[p]
      ib = ibufs[p]
      wait_load(chunk, p)
      pltpu.async_copy(fb, acc.at[ib.at[0]], ssems[p], add=True)
      pltpu.async_copy(fb, acc.at[ib.at[1]], ssems[p], add=True)
      for e in (0, 1):
        for jj in range(_CHUNK // 16):
          idxv = ib[e, pl.ds(jj * 16, 16)]
          plsc.addupdate_scatter(cntloc, [idxv], ones16)

    if not first:
      @pl.when((i - 1) * _NW + wid < _NCHUNKS)
      def _():
        wait_scatter(q)

    nchunk = (i + 2) * _NW + wid

    @pl.when(nchunk < _NCHUNKS)
    def _():
      start_load(nchunk, q)

  _it(0, 0, 2, first=True)
  _it(1, 1, 0)

  n3 = (_MAX_ITERS + 1 - 2 + 2) // 3

  @pl.loop(0, n3)
  def _(j):
    i0 = 3 * j + 2
    _it(i0, 2, 1)
    _it(i0 + 1, 0, 2)
    _it(i0 + 2, 1, 0)

  plsc.subcore_barrier()

  pltpu.sync_copy(acc.at[pl.ds(row0, _ROWS_PER_TILE)],
                  sums_hbm.at[c, pl.ds(row0, _ROWS_PER_TILE)])
  pltpu.sync_copy(cntloc, cnts_hbm.at[wid])

  @pl.when(s == _NS - 1)
  def _():
    pltpu.sync_copy(acc.at[pl.ds(tail0, _TAIL_ROWS)],
                    sums_hbm.at[c, pl.ds(tail0, _TAIL_ROWS)])


@jax.jit
def _sc_scatter(face_phi, face_node):
  mesh = plsc.VectorSubcoreMesh(core_axis_name="core",
                                subcore_axis_name="subcore")
  return pl.kernel(
      _sc_body,
      compiler_params=pltpu.CompilerParams(use_tc_tiling_on_sc=False,
                                           needs_layout_passes=False),
      out_type=[
          jax.ShapeDtypeStruct((_NC, NUM_NODES, NUM_CH), jnp.float32),
          jax.ShapeDtypeStruct((_NW, NUM_NODES), jnp.float32),
      ],
      mesh=mesh,
      scratch_types=[
          pltpu.VMEM((_CHUNK, NUM_CH), jnp.float32),
          pltpu.VMEM((_CHUNK, NUM_CH), jnp.float32),
          pltpu.VMEM((_CHUNK, NUM_CH), jnp.float32),
          pltpu.VMEM((2, _CHUNK), jnp.int32),
          pltpu.VMEM((2, _CHUNK), jnp.int32),
          pltpu.VMEM((2, _CHUNK), jnp.int32),
          pltpu.VMEM((NUM_NODES,), jnp.float32),
          pltpu.VMEM_SHARED((NUM_NODES, NUM_CH), jnp.float32),
          pltpu.SemaphoreType.DMA,
          pltpu.SemaphoreType.DMA,
          pltpu.SemaphoreType.DMA,
          pltpu.SemaphoreType.DMA,
          pltpu.SemaphoreType.DMA,
          pltpu.SemaphoreType.DMA,
      ],
  )(face_phi, face_node)


def _combine_body(s_ref, c_ref, o_ref):
  sums = s_ref[0] + s_ref[1]
  counts = jnp.sum(c_ref[...], axis=0)[:, None]
  o_ref[...] = sums / jnp.maximum(counts, 1.0)


@jax.jit
def _combine(sums, cnts):
  return pl.pallas_call(
      _combine_body,
      out_shape=jax.ShapeDtypeStruct((NUM_NODES, NUM_CH), jnp.float32),
  )(sums, cnts)


def kernel(face_phi, face_node):
  sums, cnts = _sc_scatter(face_phi, face_node)
  return _combine(sums, cnts)


# speedup vs baseline: 1.0071x; 1.0071x over previous
import jax
import jax.numpy as jnp
from jax import lax
from jax.experimental import pallas as pl
from jax.experimental.pallas import tpu as pltpu
from jax.experimental.pallas import tpu_sc as plsc

NUM_NODES = 10000
NUM_FACES = 320000
NUM_CH = 128

_NC = 2
_NS = 16
_NW = _NC * _NS
_CHUNK = 128
_NCHUNKS = NUM_FACES // _CHUNK
_ROWS_PER_TILE = 624
_TAIL_ROWS = NUM_NODES - _NS * _ROWS_PER_TILE
_MAX_ITERS = -(-_NCHUNKS // _NW)


def _sc_body(phi_hbm, fn_hbm, sums_hbm, cnts_hbm,
             fb0, fb1, ib0, ib1, cntloc, acc,
             lsem0, lsem1, ssem0, ssem1):
  c = lax.axis_index("core")
  s = lax.axis_index("subcore")
  wid = s * _NC + c

  fbufs = (fb0, fb1)
  ibufs = (ib0, ib1)
  lsems = (lsem0, lsem1)
  ssems = (ssem0, ssem1)

  z16 = jnp.zeros((16,), jnp.float32)
  ones16 = jnp.ones((16,), jnp.float32)

  base1 = (_NW + wid) * _CHUNK
  pltpu.async_copy(phi_hbm.at[pl.ds(base1, _CHUNK)], fb1, lsem1)
  pltpu.async_copy(fn_hbm.at[:, pl.ds(base1, _CHUNK)], ib1, lsem1)

  @pl.loop(0, _CHUNK)
  def _(i):
    @pl.loop(0, NUM_CH // 16)
    def _(j):
      fb0[i, pl.ds(j * 16, 16)] = z16

  @pl.loop(0, NUM_NODES // 16)
  def _(i):
    cntloc[pl.ds(i * 16, 16)] = z16

  row0 = s * _ROWS_PER_TILE
  tail0 = _NS * _ROWS_PER_TILE
  @pl.loop(0, _ROWS_PER_TILE // _CHUNK)
  def _(k):
    pltpu.sync_copy(fb0, acc.at[pl.ds(row0 + k * _CHUNK, _CHUNK)])
  rem = _ROWS_PER_TILE % _CHUNK
  if rem:
    base = row0 + (_ROWS_PER_TILE // _CHUNK) * _CHUNK
    pltpu.sync_copy(fb0.at[pl.ds(0, rem)], acc.at[pl.ds(base, rem)])

  @pl.when(s == _NS - 1)
  def _():
    pltpu.sync_copy(fb0.at[pl.ds(0, _TAIL_ROWS)],
                    acc.at[pl.ds(tail0, _TAIL_ROWS)])

  base0 = wid * _CHUNK
  pltpu.async_copy(phi_hbm.at[pl.ds(base0, _CHUNK)], fb0, lsem0)
  pltpu.async_copy(fn_hbm.at[:, pl.ds(base0, _CHUNK)], ib0, lsem0)

  plsc.subcore_barrier()

  def _iter(i, p):
    chunk = i * _NW + wid

    @pl.when(chunk < _NCHUNKS)
    def _():
      fb = fbufs[p]
      ib = ibufs[p]
      base = chunk * _CHUNK
      pltpu.make_async_copy(phi_hbm.at[pl.ds(base, _CHUNK)], fb,
                            lsems[p]).wait()
      pltpu.make_async_copy(fn_hbm.at[:, pl.ds(base, _CHUNK)], ib,
                            lsems[p]).wait()
      pltpu.async_copy(fb, acc.at[ib.at[0]], ssems[p], add=True)
      pltpu.async_copy(fb, acc.at[ib.at[1]], ssems[p], add=True)
      for e in (0, 1):
        for jj in range(_CHUNK // 16):
          idxv = ib[e, pl.ds(jj * 16, 16)]
          plsc.addupdate_scatter(cntloc, [idxv], ones16)
      pltpu.make_async_copy(fb, acc.at[ib.at[0]], ssems[p]).wait()
      pltpu.make_async_copy(fb, acc.at[ib.at[1]], ssems[p]).wait()

      nchunk = chunk + 2 * _NW

      @pl.when(nchunk < _NCHUNKS)
      def _():
        nbase = nchunk * _CHUNK
        pltpu.async_copy(phi_hbm.at[pl.ds(nbase, _CHUNK)], fb, lsems[p])
        pltpu.async_copy(fn_hbm.at[:, pl.ds(nbase, _CHUNK)], ib, lsems[p])

  @pl.loop(0, (_MAX_ITERS + 1) // 2)
  def _(j):
    _iter(2 * j, 0)
    _iter(2 * j + 1, 1)

  plsc.subcore_barrier()

  pltpu.sync_copy(acc.at[pl.ds(row0, _ROWS_PER_TILE)],
                  sums_hbm.at[c, pl.ds(row0, _ROWS_PER_TILE)])
  pltpu.sync_copy(cntloc, cnts_hbm.at[wid])

  @pl.when(s == _NS - 1)
  def _():
    pltpu.sync_copy(acc.at[pl.ds(tail0, _TAIL_ROWS)],
                    sums_hbm.at[c, pl.ds(tail0, _TAIL_ROWS)])


@jax.jit
def _sc_scatter(face_phi, face_node):
  mesh = plsc.VectorSubcoreMesh(core_axis_name="core",
                                subcore_axis_name="subcore")
  return pl.kernel(
      _sc_body,
      compiler_params=pltpu.CompilerParams(use_tc_tiling_on_sc=False,
                                           needs_layout_passes=False),
      out_type=[
          jax.ShapeDtypeStruct((_NC, NUM_NODES, NUM_CH), jnp.float32),
          jax.ShapeDtypeStruct((_NW, NUM_NODES), jnp.float32),
      ],
      mesh=mesh,
      scratch_types=[
          pltpu.VMEM((_CHUNK, NUM_CH), jnp.float32),
          pltpu.VMEM((_CHUNK, NUM_CH), jnp.float32),
          pltpu.VMEM((2, _CHUNK), jnp.int32),
          pltpu.VMEM((2, _CHUNK), jnp.int32),
          pltpu.VMEM((NUM_NODES,), jnp.float32),
          pltpu.VMEM_SHARED((NUM_NODES, NUM_CH), jnp.float32),
          pltpu.SemaphoreType.DMA,
          pltpu.SemaphoreType.DMA,
          pltpu.SemaphoreType.DMA,
          pltpu.SemaphoreType.DMA,
      ],
  )(face_phi, face_node)


def _combine_body(s_ref, c_ref, o_ref):
  sums = s_ref[0] + s_ref[1]
  counts = jnp.sum(c_ref[...], axis=0)[:, None]
  o_ref[...] = sums / jnp.maximum(counts, 1.0)


@jax.jit
def _combine(sums, cnts):
  return pl.pallas_call(
      _combine_body,
      out_shape=jax.ShapeDtypeStruct((NUM_NODES, NUM_CH), jnp.float32),
  )(sums, cnts)


def kernel(face_phi, face_node):
  sums, cnts = _sc_scatter(face_phi, face_node)
  return _combine(sums, cnts)
